# baseline (speedup 1.0000x reference)
import functools

import jax
import jax.numpy as jnp
from jax import lax
from jax.experimental import pallas as pl
from jax.experimental.pallas import tpu as pltpu
from jax.experimental.pallas import tpu_sc as plsc

B = 16384
N_NUM = 13
N_CAT = 26
CARD1 = 100001
D = 32
N_FEAT = N_NUM + N_CAT

NC = 2
NS = 16
NWORK = NC * NS
BPT = B // NWORK
G = 32
NCHUNK = BPT // G
R = G * N_CAT
L = 16


def _tokenize_body(xcat_hbm, table_hbm, lnb_hbm, numw_hbm, numb_hbm,
                   femb_hbm, out_hbm,
                   xc_v, idx_v, rows_v, full_v, lnb_v, nw_v, nb_v, fe_v, sem):
    c = lax.axis_index("c")
    s = lax.axis_index("s")
    wid = s * NC + c

    pltpu.sync_copy(lnb_hbm, lnb_v)
    pltpu.sync_copy(numw_hbm, nw_v)
    pltpu.sync_copy(numb_hbm, nb_v)
    pltpu.sync_copy(femb_hbm, fe_v)

    def _num(g, _):
        for j in range(N_NUM):
            for h in range(2):
                sl = pl.ds(h * L, L)
                full_v[g, j, sl] = (lnb_v[j, sl] * nw_v[j, sl] + nb_v[j, sl]
                                    + fe_v[j, sl])
        return 0
    lax.fori_loop(0, G, _num, 0)

    base_row = wid * BPT

    def _chunk(ci, _):
        b0 = base_row + ci * G
        pltpu.sync_copy(xcat_hbm.at[pl.ds(b0 * N_CAT, R)], xc_v)

        def _idx(v, _):
            pos = v * L + lax.iota(jnp.int32, L)
            f = lax.rem(pos, N_CAT)
            idx_v[pl.ds(v * L, L)] = xc_v[pl.ds(v * L, L)] + f * CARD1
            return 0
        lax.fori_loop(0, R // L, _idx, 0, unroll=4)

        pltpu.async_copy(table_hbm.at[idx_v], rows_v, sem).wait()

        def _add(g, _):
            for j in range(N_CAT):
                for h in range(2):
                    sl = pl.ds(h * L, L)
                    full_v[g, N_NUM + j, sl] = (rows_v[g * N_CAT + j, sl]
                                                + fe_v[N_NUM + j, sl])
            return 0
        lax.fori_loop(0, G, _add, 0)

        pltpu.sync_copy(full_v, out_hbm.at[pl.ds(b0, G)])
        return 0

    lax.fori_loop(0, NCHUNK, _chunk, 0)


_tokenize = functools.partial(
    pl.kernel,
    out_type=jax.ShapeDtypeStruct((B, N_FEAT, D), jnp.float32),
    mesh=plsc.VectorSubcoreMesh(core_axis_name="c", subcore_axis_name="s"),
    scratch_types=[
        pltpu.VMEM((R,), jnp.int32),
        pltpu.VMEM((R,), jnp.int32),
        pltpu.VMEM((R, D), jnp.float32),
        pltpu.VMEM((G, N_FEAT, D), jnp.float32),
        pltpu.VMEM((N_NUM, D), jnp.float32),
        pltpu.VMEM((N_NUM, D), jnp.float32),
        pltpu.VMEM((N_NUM, D), jnp.float32),
        pltpu.VMEM((N_FEAT, D), jnp.float32),
        pltpu.SemaphoreType.DMA,
    ],
    compiler_params=pltpu.CompilerParams(use_tc_tiling_on_sc=False),
)(_tokenize_body)


def kernel(x_num, x_cat, ln_g, ln_b, num_w, num_b, cat_tables, feat_emb):
    del x_num, ln_g
    xcat_flat = x_cat.reshape(B * N_CAT)
    table_flat = cat_tables.reshape(N_CAT * CARD1, D)
    lnb32 = jnp.broadcast_to(ln_b, (N_NUM, D))
    return _tokenize(xcat_flat, table_flat, lnb32, num_w, num_b, feat_emb)


# speedup vs baseline: 4.1951x; 4.1951x over previous
import functools

import jax
import jax.numpy as jnp
from jax import lax
from jax.experimental import pallas as pl
from jax.experimental.pallas import tpu as pltpu
from jax.experimental.pallas import tpu_sc as plsc

B = 16384
N_NUM = 13
N_CAT = 26
CARD1 = 100001
D = 32
N_FEAT = N_NUM + N_CAT

NC = 2
NS = 16
NWORK = NC * NS
L = 16

CH = 1024
MAIN = 99328
NSLAB = MAIN // CH
TAIL = CARD1 - MAIN
NITEM = N_CAT * NSLAB
TFLAT_ROWS = N_CAT * MAIN

BPT = B // NWORK
G = 32
NCHUNK = BPT // G
R = G * N_CAT


def _detile_body(tbl_hbm, xct_hbm, tflat_hbm, xcf_hbm,
                 lanes_v, rowm_v, xrow_v, sems):
    c = lax.axis_index("c")
    s = lax.axis_index("s")
    wid = s * NC + c

    @pl.when(wid < N_CAT)
    def _():
        pltpu.sync_copy(xct_hbm.at[wid], xrow_v)
        pltpu.sync_copy(xrow_v, xcf_hbm.at[pl.ds(wid * B, B)])

    ib1024 = lax.iota(jnp.int32, L) * CH

    def _item(k, _):
        it = wid + k * NWORK

        @pl.when(it < NITEM)
        def _():
            f = it // NSLAB
            i0 = (it % NSLAB) * CH
            descs = []
            for cc in range(D):
                descs.append(pltpu.async_copy(
                    tbl_hbm.at[f, cc, pl.ds(i0, CH)],
                    lanes_v.at[pl.ds(cc * CH, CH)], sems))
            for d in descs:
                d.wait()

            def _tp(v, _):
                i = v >> 1
                half = (v & 1) << 4
                idx = ib1024 + (half * CH + i)
                rowm_v[pl.ds(v * L, L)] = plsc.load_gather(lanes_v, [idx])
                return 0
            lax.fori_loop(0, (CH * D) // L, _tp, 0, unroll=4)

            pltpu.sync_copy(
                rowm_v, tflat_hbm.at[pl.ds((f * MAIN + i0) * D, CH * D)])
        return 0

    lax.fori_loop(0, (NITEM + NWORK - 1) // NWORK, _item, 0)


_detile = functools.partial(
    pl.kernel,
    out_type=(jax.ShapeDtypeStruct((TFLAT_ROWS * D,), jnp.float32),
              jax.ShapeDtypeStruct((B * N_CAT,), jnp.int32)),
    mesh=plsc.VectorSubcoreMesh(core_axis_name="c", subcore_axis_name="s"),
    scratch_types=[
        pltpu.VMEM((D * CH,), jnp.float32),
        pltpu.VMEM((CH * D,), jnp.float32),
        pltpu.VMEM((B,), jnp.int32),
        pltpu.SemaphoreType.DMA,
    ],
    compiler_params=pltpu.CompilerParams(use_tc_tiling_on_sc=True,
                                         needs_layout_passes=False),
)(_detile_body)


def _gather_body(tflat_hbm, tail_hbm, xcf_hbm, lnb_hbm, numw_hbm, numb_hbm,
                 femb_hbm, out_hbm,
                 ids_v, idx_v, idxt_v, rows_v, rowst_v, obuf_v,
                 lnb_v, nw_v, nb_v, fe_v, numc_v, sem):
    c = lax.axis_index("c")
    s = lax.axis_index("s")
    wid = s * NC + c

    pltpu.sync_copy(lnb_hbm, lnb_v)
    pltpu.sync_copy(numw_hbm, nw_v)
    pltpu.sync_copy(numb_hbm, nb_v)
    pltpu.sync_copy(femb_hbm, fe_v)

    iota = lax.iota(jnp.int32, L)

    def _numc(v, _):
        pos = v * L + iota
        sl = pl.ds(v * L, L)
        lnb16 = plsc.load_gather(lnb_v, [pos >> 5])
        numc_v[sl] = lnb16 * nw_v[sl] + nb_v[sl] + fe_v[sl]
        return 0
    lax.fori_loop(0, (N_NUM * D) // L, _numc, 0)

    def _numfill(j, _):
        for tc in range(4):
            for cr in range(8):
                val = plsc.load_gather(
                    numc_v, [jnp.full((L,), j * D + tc * 8 + cr, jnp.int32)])
                for brv in range(G // L):
                    obuf_v[j, tc, cr, pl.ds(brv * L, L)] = val
        return 0
    lax.fori_loop(0, N_NUM, _numfill, 0)

    iota32 = iota * D

    def _chunk(ci, _):
        b0 = wid * BPT + ci * G
        tb = b0 // 128
        br0 = b0 % 128

        for f in range(N_CAT):
            pltpu.sync_copy(xcf_hbm.at[pl.ds(f * B + b0, G)],
                            ids_v.at[pl.ds(f * G, G)])

        def _idx(v, _):
            pos = v * L + iota
            f = pos >> 5
            sl = pl.ds(v * L, L)
            ids16 = ids_v[sl]
            idx_v[sl] = jnp.minimum(ids16, MAIN - 1) + f * MAIN
            idxt_v[sl] = jnp.maximum(ids16 - MAIN, 0) + f * TAIL
            return 0
        lax.fori_loop(0, R // L, _idx, 0, unroll=4)

        pltpu.async_copy(tflat_hbm.at[idx_v], rows_v, sem).wait()
        pltpu.async_copy(tail_hbm.at[idxt_v], rowst_v, sem).wait()

        def _perm(f, _):
            fbase = f * G
            for tc in range(4):
                for cr in range(8):
                    col = tc * 8 + cr
                    fev = plsc.load_gather(
                        fe_v,
                        [jnp.full((L,), (N_NUM + f) * D + col, jnp.int32)])
                    for brv in range(G // L):
                        ridx = fbase + brv * L + iota
                        cidx = jnp.full((L,), col, jnp.int32)
                        vmain = plsc.load_gather(rows_v, [ridx, cidx])
                        vtail = plsc.load_gather(rowst_v, [ridx, cidx])
                        ids16 = ids_v[pl.ds(fbase + brv * L, L)]
                        val = jnp.where(ids16 >= MAIN, vtail, vmain) + fev
                        obuf_v[N_NUM + f, tc, cr, pl.ds(brv * L, L)] = val
            return 0
        lax.fori_loop(0, N_CAT, _perm, 0)

        pltpu.sync_copy(obuf_v,
                        out_hbm.at[:, :, tb, :, pl.ds(br0, G)])
        return 0

    lax.fori_loop(0, NCHUNK, _chunk, 0)


_gather = functools.partial(
    pl.kernel,
    out_type=jax.ShapeDtypeStruct((N_FEAT, 4, B // 128, 8, 128), jnp.float32),
    mesh=plsc.VectorSubcoreMesh(core_axis_name="c", subcore_axis_name="s"),
    scratch_types=[
        pltpu.VMEM((R,), jnp.int32),
        pltpu.VMEM((R,), jnp.int32),
        pltpu.VMEM((R,), jnp.int32),
        pltpu.VMEM((R, D), jnp.float32),
        pltpu.VMEM((R, D), jnp.float32),
        pltpu.VMEM((N_FEAT, 4, 8, G), jnp.float32),
        pltpu.VMEM((16,), jnp.float32),
        pltpu.VMEM((N_NUM * D,), jnp.float32),
        pltpu.VMEM((N_NUM * D,), jnp.float32),
        pltpu.VMEM((N_FEAT * D,), jnp.float32),
        pltpu.VMEM((N_NUM * D,), jnp.float32),
        pltpu.SemaphoreType.DMA,
    ],
    compiler_params=pltpu.CompilerParams(use_tc_tiling_on_sc=False,
                                         needs_layout_passes=False),
)(_gather_body)


def kernel(x_num, x_cat, ln_g, ln_b, num_w, num_b, cat_tables, feat_emb):
    del x_num, ln_g
    tbl_t = cat_tables.transpose(0, 2, 1)
    xct = x_cat.T
    tflat, xcf = _detile(tbl_t, xct)
    tail = cat_tables[:, MAIN:, :].reshape(N_CAT * TAIL, D)
    lnb16 = jnp.pad(ln_b.reshape(N_NUM), (0, 16 - N_NUM))
    out5 = _gather(tflat.reshape(TFLAT_ROWS, D), tail, xcf, lnb16,
                   num_w.reshape(N_NUM * D), num_b.reshape(N_NUM * D),
                   feat_emb.reshape(N_FEAT * D))
    return out5.transpose(2, 4, 0, 1, 3).reshape(B, N_FEAT, D)


# speedup vs baseline: 7.1999x; 1.7163x over previous
import functools

import jax
import jax.numpy as jnp
from jax import lax
from jax.experimental import pallas as pl
from jax.experimental.pallas import tpu as pltpu
from jax.experimental.pallas import tpu_sc as plsc

B = 16384
N_NUM = 13
N_CAT = 26
CARD1 = 100001
D = 32
N_FEAT = N_NUM + N_CAT

NC = 2
NS = 16
NWORK = NC * NS
L = 16

CH = 1024
NSLAB = 97
SFULL = NSLAB * CH
TAILN = CARD1 - SFULL
NITEM = N_CAT * NSLAB

BPT = B // NWORK
G = 32
NCHUNK = BPT // G
R = G * N_CAT
RP = 33


def _detile_body(tbl_hbm, xct_hbm, tailflat_hbm, tflat_hbm, xcf_hbm,
                 lanes_v, packed_v, xrow_v, xcbuf_v, sems):
    c = lax.axis_index("c")
    s = lax.axis_index("s")
    wid = s * NC + c

    iota = lax.iota(jnp.int32, L)

    def _xc(f, _):
        pltpu.sync_copy(xct_hbm.at[f, pl.ds(wid * BPT, BPT)], xrow_v)

        def _pack(v, _):
            off = (v >> 1) * (N_CAT * G) + f * G + (v & 1) * L
            xcbuf_v[pl.ds(off, L)] = xrow_v[pl.ds(v * L, L)]
            return 0
        lax.fori_loop(0, BPT // L, _pack, 0, unroll=4)
        return 0
    lax.fori_loop(0, N_CAT, _xc, 0)
    pltpu.sync_copy(xcbuf_v, xcf_hbm.at[pl.ds(wid * BPT * N_CAT, BPT * N_CAT)])

    def _diag_transpose(n, stride):
        nv = (n + L - 1) // L

        def _c0(c0, _):
            cd = lax.rem(c0 + iota, D)
            srcb = cd * stride
            dstb = iota * D + cd

            def _i0(v, _):
                i = v * L + iota
                icl = jnp.minimum(i, n - 1)
                vals = plsc.load_gather(lanes_v, [srcb + icl])
                plsc.store_scatter(packed_v, [dstb + v * (L * D)], vals,
                                   mask=i < n)
                return 0
            lax.fori_loop(0, nv, _i0, 0, unroll=4)
            return 0
        lax.fori_loop(0, D, _c0, 0)

    def _item(k, _):
        it = wid + k * NWORK

        @pl.when(it < NITEM)
        def _():
            f = it // NSLAB
            i0 = (it % NSLAB) * CH
            descs = []
            for cc in range(D):
                descs.append(pltpu.async_copy(
                    tbl_hbm.at[f, cc, pl.ds(i0, CH)],
                    lanes_v.at[pl.ds(cc * CH, CH)], sems))
            for d in descs:
                d.wait()
            _diag_transpose(CH, CH)
            pltpu.sync_copy(
                packed_v, tflat_hbm.at[pl.ds((f * CARD1 + i0) * D, CH * D)])
        return 0

    lax.fori_loop(0, (NITEM + NWORK - 1) // NWORK, _item, 0)

    @pl.when(wid < N_CAT)
    def _():
        f = wid
        nt = TAILN * D
        pltpu.sync_copy(tailflat_hbm.at[pl.ds(f * nt, nt)],
                        lanes_v.at[pl.ds(0, nt)])
        pltpu.sync_copy(lanes_v.at[pl.ds(0, nt)],
                        tflat_hbm.at[pl.ds((f * CARD1 + SFULL) * D, nt)])


_detile = functools.partial(
    pl.kernel,
    out_type=(jax.ShapeDtypeStruct((N_CAT * CARD1 * D,), jnp.float32),
              jax.ShapeDtypeStruct((B * N_CAT,), jnp.int32)),
    mesh=plsc.VectorSubcoreMesh(core_axis_name="c", subcore_axis_name="s"),
    scratch_types=[
        pltpu.VMEM((D * CH,), jnp.float32),
        pltpu.VMEM((CH * D,), jnp.float32),
        pltpu.VMEM((BPT,), jnp.int32),
        pltpu.VMEM((BPT * N_CAT,), jnp.int32),
        pltpu.SemaphoreType.DMA,
    ],
    compiler_params=pltpu.CompilerParams(use_tc_tiling_on_sc=True,
                                         needs_layout_passes=False),
)(_detile_body)


def _gather_body(tflat_hbm, xcf_hbm, lnb_hbm, numw_hbm, numb_hbm,
                 femb_hbm, out_hbm,
                 ids_v, idx_v, rows_v, obuf_v,
                 lnb_v, nw_v, nb_v, fe_v, numc_v, sem):
    c = lax.axis_index("c")
    s = lax.axis_index("s")
    wid = s * NC + c

    pltpu.sync_copy(lnb_hbm, lnb_v)
    pltpu.sync_copy(numw_hbm, nw_v)
    pltpu.sync_copy(numb_hbm, nb_v)
    pltpu.sync_copy(femb_hbm, fe_v)

    iota = lax.iota(jnp.int32, L)

    def _numc(v, _):
        pos = v * L + iota
        sl = pl.ds(v * L, L)
        lnb16 = plsc.load_gather(lnb_v, [pos >> 5])
        numc_v[sl] = lnb16 * nw_v[sl] + nb_v[sl] + fe_v[sl]
        return 0
    lax.fori_loop(0, (N_NUM * D) // L, _numc, 0)

    def _numfill(j, _):
        for h in range(2):
            cvec = h * L + iota
            tc16 = cvec >> 3
            cr16 = cvec & 7
            val = numc_v[pl.ds(j * D + h * L, L)]

            def _g(g, _):
                plsc.store_scatter(
                    obuf_v,
                    [jnp.full((L,), j, jnp.int32), tc16, cr16,
                     jnp.full((L,), g, jnp.int32)], val)
                return 0
            lax.fori_loop(0, G, _g, 0, unroll=4)
        return 0
    lax.fori_loop(0, N_NUM, _numfill, 0)

    def _chunk(ci, _):
        b0 = wid * BPT + ci * G
        tb = b0 // 128
        brh = (b0 % 128) // G

        pltpu.sync_copy(xcf_hbm.at[pl.ds(wid * BPT * N_CAT + ci * R, R)],
                        ids_v)

        def _idx(v, _):
            pos = v * L + iota
            sl = pl.ds(v * L, L)
            idx_v[sl] = ids_v[sl] + (pos >> 5) * CARD1
            return 0
        lax.fori_loop(0, R // L, _idx, 0, unroll=4)

        pltpu.async_copy(tflat_hbm.at[idx_v], rows_v, sem).wait()

        def _perm(f, _):
            j16 = jnp.full((L,), N_NUM + f, jnp.int32)
            for h in range(2):
                cvec = h * L + iota
                tc16 = cvec >> 3
                cr16 = cvec & 7
                fev = fe_v[pl.ds((N_NUM + f) * D + h * L, L)]

                def _g(g, _):
                    val = rows_v[f * G + g, pl.ds(h * L, L)] + fev
                    plsc.store_scatter(
                        obuf_v,
                        [j16, tc16, cr16, jnp.full((L,), g, jnp.int32)],
                        val)
                    return 0
                lax.fori_loop(0, G, _g, 0, unroll=4)
            return 0
        lax.fori_loop(0, N_CAT, _perm, 0)

        pltpu.sync_copy(obuf_v.at[:, :, :, pl.ds(0, G)],
                        out_hbm.at[:, :, tb, :, brh, :])
        return 0

    lax.fori_loop(0, NCHUNK, _chunk, 0)


_gather = functools.partial(
    pl.kernel,
    out_type=jax.ShapeDtypeStruct((N_FEAT, 4, B // 128, 8, 128 // G, G),
                                  jnp.float32),
    mesh=plsc.VectorSubcoreMesh(core_axis_name="c", subcore_axis_name="s"),
    scratch_types=[
        pltpu.VMEM((R,), jnp.int32),
        pltpu.VMEM((R,), jnp.int32),
        pltpu.VMEM((R, D), jnp.float32),
        pltpu.VMEM((N_FEAT, 4, 8, RP), jnp.float32),
        pltpu.VMEM((16,), jnp.float32),
        pltpu.VMEM((N_NUM * D,), jnp.float32),
        pltpu.VMEM((N_NUM * D,), jnp.float32),
        pltpu.VMEM((N_FEAT * D,), jnp.float32),
        pltpu.VMEM((N_NUM * D,), jnp.float32),
        pltpu.SemaphoreType.DMA,
    ],
    compiler_params=pltpu.CompilerParams(use_tc_tiling_on_sc=False,
                                         needs_layout_passes=False),
)(_gather_body)


def kernel(x_num, x_cat, ln_g, ln_b, num_w, num_b, cat_tables, feat_emb):
    del x_num, ln_g
    tbl_t = cat_tables.transpose(0, 2, 1)
    xct = x_cat.T
    tailflat = cat_tables[:, SFULL:, :].reshape(N_CAT * TAILN * D)
    tflat, xcf = _detile(tbl_t, xct, tailflat)
    lnb16 = jnp.pad(ln_b.reshape(N_NUM), (0, 16 - N_NUM))
    out6 = _gather(tflat.reshape(N_CAT * CARD1, D), xcf, lnb16,
                   num_w.reshape(N_NUM * D), num_b.reshape(N_NUM * D),
                   feat_emb.reshape(N_FEAT * D))
    return out6.transpose(2, 4, 5, 0, 1, 3).reshape(B, N_FEAT, D)
